# Initial kernel scaffold; baseline (speedup 1.0000x reference)
import jax
import jax.numpy as jnp
from jax.experimental import pallas as pl


def kernel(xyz, params):
    raise NotImplementedError("write your pallas kernel here")


# speedup vs baseline: 1.0273x; 1.0273x over previous
import functools

import jax
import jax.numpy as jnp
from jax.experimental import pallas as pl


def _cumsum_lanes(x):
    n = x.shape[-1]
    s = 1
    while s < n:
        x = x + jnp.pad(x[:, :-s], ((0, 0), (s, 0)))
        s *= 2
    return x


def _fps_body(xT_ref, out_ref, *, npoint):
    x = xT_ref[...]
    b, _, n = x.shape
    iota = jax.lax.broadcasted_iota(jnp.int32, (b, n), 1)

    def body(i, state):
        dist, oh = state
        c = jnp.sum(x * oh[:, None, :], axis=2, keepdims=True)
        out_ref[pl.ds(i, 1), :, :] = jnp.transpose(c, (2, 0, 1))
        d = jnp.sum((x - c) ** 2, axis=1)
        dist = jnp.minimum(dist, d)
        m = jnp.max(dist, axis=1, keepdims=True)
        eq = dist == m
        first = jnp.min(jnp.where(eq, iota, n), axis=1, keepdims=True)
        oh_new = (iota == first).astype(jnp.float32)
        return dist, oh_new

    dist0 = jnp.full((b, n), 1e10, jnp.float32)
    oh0 = (iota == 0).astype(jnp.float32)
    jax.lax.fori_loop(0, npoint, body, (dist0, oh0))


def _fps(xT, npoint):
    xyz = jnp.transpose(xT, (0, 2, 1))
    n = xyz.shape[1]

    def single(x):
        def body(i, state):
            centroids, distance, farthest = state
            centroids = centroids.at[i].set(farthest)
            centroid = x[farthest]
            d = jnp.sum((x - centroid) ** 2, -1)
            distance = jnp.minimum(distance, d)
            farthest = jnp.argmax(distance).astype(jnp.int32)
            return (centroids, distance, farthest)
        init = (jnp.zeros((npoint,), dtype=jnp.int32),
                jnp.full((n,), 1e10, dtype=jnp.float32), jnp.int32(0))
        return jax.lax.fori_loop(0, npoint, body, init)[0]

    idx = jax.vmap(single)(xyz)
    return jax.vmap(lambda p, i: p[i])(xyz, idx)


def _group_body(idx_ref, nx_ref, xf_ref, out_ref, *, k, nfeat):
    idx = idx_ref[0]
    nx = nx_ref[0]
    xf = xf_ref[0]
    n = xf.shape[0]
    lane = jax.lax.broadcasted_iota(jnp.int32, (idx.shape[0], n), 1)
    for kk in range(k):
        sel = (lane == idx[:, kk:kk + 1]).astype(jnp.float32)
        g = jnp.dot(sel, xf, precision=jax.lax.Precision.HIGHEST,
                    preferred_element_type=jnp.float32)
        gx = g[:, :3] - nx
        row = jnp.concatenate([gx, g[:, 3:]], axis=-1) if nfeat else gx
        out_ref[0, :, kk, :] = row


def _sqdist_xla(src, dst):
    d = -2.0 * jnp.matmul(src, jnp.swapaxes(dst, 1, 2))
    d = d + jnp.sum(src ** 2, -1)[:, :, None]
    d = d + jnp.sum(dst ** 2, -1)[:, None, :]
    return d


def _group(new_xyz, xyz, xf, radius, k, sblk):
    b, s, _ = new_xyz.shape
    n = xyz.shape[1]
    c = xf.shape[2]
    nfeat = c > 3
    sqrdists = _sqdist_xla(new_xyz, xyz)
    idx = jnp.broadcast_to(jnp.arange(n, dtype=jnp.int32), (b, s, n))
    idx = jnp.where(sqrdists > radius * radius, n, idx)
    idx = jnp.sort(idx, axis=-1)[:, :, :k]
    first = idx[:, :, 0:1]
    idx = jnp.where(idx == n, jnp.broadcast_to(first, idx.shape), idx)
    g = jax.vmap(lambda p, i: p[i])(xf, idx)
    gx = g[..., :3] - new_xyz[:, :, None, :]
    return jnp.concatenate([gx, g[..., 3:]], axis=-1) if nfeat else gx


def _layer_body(x_ref, a_ref, c_ref, wt_ref, b_ref, y_ref, s1_ref, s2_ref,
                *, norm_in):
    x = x_ref[...]
    if norm_in:
        x = jnp.maximum(x * a_ref[...] + c_ref[...], 0.0)
    y = jnp.dot(x, wt_ref[...], preferred_element_type=jnp.float32)
    y = y + b_ref[...]
    y_ref[...] = y
    ps = jnp.sum(y, axis=0, keepdims=True)
    pq = jnp.sum(y * y, axis=0, keepdims=True)

    @pl.when(pl.program_id(0) == 0)
    def _():
        s1_ref[...] = ps
        s2_ref[...] = pq

    @pl.when(pl.program_id(0) != 0)
    def _():
        s1_ref[...] += ps
        s2_ref[...] += pq


def _layer(x, ac, wt, bias):
    p, cin = x.shape
    cout = wt.shape[1]
    blk = min(p, 8192 if cin <= 128 else 2048)
    grid = (p // blk,)
    norm_in = ac is not None
    if ac is None:
        ac = (jnp.ones((1, cin), jnp.float32), jnp.zeros((1, cin), jnp.float32))
    fullspec = pl.BlockSpec((1, cin), lambda i: (0, 0))
    return pl.pallas_call(
        functools.partial(_layer_body, norm_in=norm_in),
        grid=grid,
        in_specs=[
            pl.BlockSpec((blk, cin), lambda i: (i, 0)),
            fullspec, fullspec,
            pl.BlockSpec((cin, cout), lambda i: (0, 0)),
            pl.BlockSpec((1, cout), lambda i: (0, 0)),
        ],
        out_specs=[
            pl.BlockSpec((blk, cout), lambda i: (i, 0)),
            pl.BlockSpec((1, cout), lambda i: (0, 0)),
            pl.BlockSpec((1, cout), lambda i: (0, 0)),
        ],
        out_shape=[
            jax.ShapeDtypeStruct((p, cout), jnp.float32),
            jax.ShapeDtypeStruct((1, cout), jnp.float32),
            jax.ShapeDtypeStruct((1, cout), jnp.float32),
        ],
    )(x, ac[0], ac[1], wt, bias)


def _ac_from_stats(s1, s2, p, g, bt):
    m = s1[0] / p
    v = s2[0] / p - m * m
    inv = g / jnp.sqrt(v + 1e-5)
    return inv.reshape(1, -1), (bt - m * inv).reshape(1, -1)


def _mlp_chain(x, ps):
    ac = None
    y = x
    for (w, b, g, bt) in ps:
        y, s1, s2 = _layer(y, ac, w.T, b.reshape(1, -1))
        ac = _ac_from_stats(s1, s2, float(y.shape[0]), g, bt)
    return y, ac


def _norm_max_body(y_ref, a_ref, c_ref, o_ref):
    z = jnp.maximum(y_ref[...] * a_ref[...] + c_ref[...], 0.0)
    o_ref[...] = jnp.max(z, axis=1)


def _norm_relu_max(y3, ac):
    p, k, c = y3.shape
    blk = min(p, 512)
    return pl.pallas_call(
        _norm_max_body,
        grid=(p // blk,),
        in_specs=[
            pl.BlockSpec((blk, k, c), lambda i: (i, 0, 0)),
            pl.BlockSpec((1, 1, c), lambda i: (0, 0, 0)),
            pl.BlockSpec((1, 1, c), lambda i: (0, 0, 0)),
        ],
        out_specs=pl.BlockSpec((blk, c), lambda i: (i, 0)),
        out_shape=jax.ShapeDtypeStruct((p, c), jnp.float32),
    )(y3, ac[0].reshape(1, 1, c), ac[1].reshape(1, 1, c))


def _norm_body(y_ref, a_ref, c_ref, o_ref):
    o_ref[...] = jnp.maximum(y_ref[...] * a_ref[...] + c_ref[...], 0.0)


def _norm_relu(y, ac):
    p, c = y.shape
    blk = min(p, 4096)
    return pl.pallas_call(
        _norm_body,
        grid=(p // blk,),
        in_specs=[
            pl.BlockSpec((blk, c), lambda i: (i, 0)),
            pl.BlockSpec((1, c), lambda i: (0, 0)),
            pl.BlockSpec((1, c), lambda i: (0, 0)),
        ],
        out_specs=pl.BlockSpec((blk, c), lambda i: (i, 0)),
        out_shape=jax.ShapeDtypeStruct((p, c), jnp.float32),
    )(y, ac[0], ac[1])


def _interp_body(x1_ref, x2T_ref, p2_ref, o_ref):
    x1 = x1_ref[0]
    x2T = x2T_ref[0]
    d = -2.0 * jnp.dot(x1, x2T, preferred_element_type=jnp.float32)
    ss = (x1[:, 0:1] ** 2 + x1[:, 1:2] ** 2) + x1[:, 2:3] ** 2
    d = d + ss
    d = d + jnp.sum(x2T ** 2, axis=0, keepdims=True)
    wacc = jnp.zeros_like(d)
    wsum = jnp.zeros((d.shape[0], 1), jnp.float32)
    for _ in range(3):
        m = jnp.min(d, axis=1, keepdims=True)
        eq = d == m
        fm = jnp.logical_and(eq, _cumsum_lanes(eq.astype(jnp.float32)) == 1.0)
        w = 1.0 / jnp.maximum(m, 1e-10)
        wacc = wacc + w * fm.astype(jnp.float32)
        wsum = wsum + w
        d = jnp.where(fm, 1e30, d)
    wacc = wacc / wsum
    o_ref[0] = jnp.dot(wacc, p2_ref[0], precision=jax.lax.Precision.HIGHEST,
                       preferred_element_type=jnp.float32)


def _interp(xyz1, xyz2, p2, blk):
    dists = _sqdist_xla(xyz1, xyz2)
    neg, idx = jax.lax.top_k(-dists, 3)
    d3 = jnp.maximum(-neg, 1e-10)
    weight = 1.0 / d3
    weight = weight / jnp.sum(weight, axis=-1, keepdims=True)
    return jnp.sum(jax.vmap(lambda p, i: p[i])(p2, idx) * weight[..., None],
                   axis=2)


def _interp_pallas(xyz1, xyz2, p2, blk):
    b, n1, _ = xyz1.shape
    n2 = xyz2.shape[1]
    c2 = p2.shape[2]
    x2T = jnp.transpose(xyz2, (0, 2, 1))
    return pl.pallas_call(
        _interp_body,
        grid=(b, n1 // blk),
        in_specs=[
            pl.BlockSpec((1, blk, 3), lambda i, j: (i, j, 0)),
            pl.BlockSpec((1, 3, n2), lambda i, j: (i, 0, 0)),
            pl.BlockSpec((1, n2, c2), lambda i, j: (i, 0, 0)),
        ],
        out_specs=pl.BlockSpec((1, blk, c2), lambda i, j: (i, j, 0)),
        out_shape=jax.ShapeDtypeStruct((b, n1, c2), jnp.float32),
    )(xyz1, x2T, p2)


def _head_body(y_ref, a_ref, c_ref, wt_ref, b_ref, o_ref):
    z = jnp.maximum(y_ref[...] * a_ref[...] + c_ref[...], 0.0)
    lg = jnp.dot(z, wt_ref[...], preferred_element_type=jnp.float32)
    lg = lg + b_ref[...]
    mx = jnp.max(lg, axis=-1, keepdims=True)
    sh = lg - mx
    lse = jnp.log(jnp.sum(jnp.exp(sh), axis=-1, keepdims=True))
    o_ref[...] = sh - lse


def _head_out(y, ac, w2, b2):
    p, c = y.shape
    nc = w2.shape[0]
    blk = min(p, 2048)
    return pl.pallas_call(
        _head_body,
        grid=(p // blk,),
        in_specs=[
            pl.BlockSpec((blk, c), lambda i: (i, 0)),
            pl.BlockSpec((1, c), lambda i: (0, 0)),
            pl.BlockSpec((1, c), lambda i: (0, 0)),
            pl.BlockSpec((c, nc), lambda i: (0, 0)),
            pl.BlockSpec((1, nc), lambda i: (0, 0)),
        ],
        out_specs=pl.BlockSpec((blk, nc), lambda i: (i, 0)),
        out_shape=jax.ShapeDtypeStruct((p, nc), jnp.float32),
    )(y, ac[0], ac[1], w2.T, b2.reshape(1, -1))


def _set_abstraction(xyz, points, npoint, radius, k, ps, sblk):
    b, n, _ = xyz.shape
    xT = jnp.transpose(xyz, (0, 2, 1))
    new_xyz = _fps(xT, npoint)
    xf = xyz if points is None else jnp.concatenate([xyz, points], axis=-1)
    grouped = _group(new_xyz, xyz, xf, radius, k, sblk)
    c = grouped.shape[-1]
    y, ac = _mlp_chain(grouped.reshape(b * npoint * k, c), ps)
    cout = y.shape[-1]
    out = _norm_relu_max(y.reshape(b * npoint, k, cout), ac)
    return new_xyz, out.reshape(b, npoint, cout)


def _feature_prop(xyz1, xyz2, p1, p2, ps, blk):
    b, n1, _ = xyz1.shape
    interp = _interp(xyz1, xyz2, p2, blk)
    x = interp if p1 is None else jnp.concatenate([p1, interp], axis=-1)
    c = x.shape[-1]
    y, ac = _mlp_chain(x.reshape(b * n1, c), ps)
    cout = y.shape[-1]
    return _norm_relu(y, ac).reshape(b, n1, cout)


def kernel(xyz, params):
    b = xyz.shape[0]
    n = xyz.shape[2]
    l0_xyz = jnp.transpose(xyz, (0, 2, 1))
    l1_xyz, l1_p = _set_abstraction(l0_xyz, None, 1024, 0.1, 32,
                                    params['sa1'], sblk=256)
    l2_xyz, l2_p = _set_abstraction(l1_xyz, l1_p, 256, 0.2, 32,
                                    params['sa2'], sblk=256)
    l3_xyz, l3_p = _set_abstraction(l2_xyz, l2_p, 64, 0.4, 32,
                                    params['sa3'], sblk=64)
    l4_xyz, l4_p = _set_abstraction(l3_xyz, l3_p, 16, 0.8, 32,
                                    params['sa4'], sblk=16)
    l3_p = _feature_prop(l3_xyz, l4_xyz, l3_p, l4_p, params['fp4'], blk=64)
    l2_p = _feature_prop(l2_xyz, l3_xyz, l2_p, l3_p, params['fp3'], blk=256)
    l1_p = _feature_prop(l1_xyz, l2_xyz, l1_p, l2_p, params['fp2'], blk=512)
    l0_p = _feature_prop(l0_xyz, l1_xyz, None, l1_p, params['fp1'], blk=512)
    y, ac = _mlp_chain(l0_p.reshape(b * n, -1), params['head'])
    w2, b2 = params['conv2']
    out = _head_out(y, ac, w2, b2)
    return out.reshape(b, n, -1)


# speedup vs baseline: 1.1075x; 1.0781x over previous
import functools

import jax
import jax.numpy as jnp
from jax.experimental import pallas as pl


def _cumsum_lanes(x):
    n = x.shape[-1]
    s = 1
    while s < n:
        x = x + jnp.pad(x[:, :-s], ((0, 0), (s, 0)))
        s *= 2
    return x


def _fps_body(xT_ref, out_ref, *, npoint):
    x = xT_ref[...]
    b, _, n = x.shape
    iota = jax.lax.broadcasted_iota(jnp.int32, (b, n), 1)

    def body(i, state):
        dist, oh = state
        c = jnp.sum(x * oh[:, None, :], axis=2, keepdims=True)
        out_ref[pl.ds(i, 1), :, :] = jnp.transpose(c, (2, 0, 1))
        d = jnp.sum((x - c) ** 2, axis=1)
        dist = jnp.minimum(dist, d)
        m = jnp.max(dist, axis=1, keepdims=True)
        eq = dist == m
        first = jnp.min(jnp.where(eq, iota, n), axis=1, keepdims=True)
        oh_new = (iota == first).astype(jnp.float32)
        return dist, oh_new

    dist0 = jnp.full((b, n), 1e10, jnp.float32)
    oh0 = (iota == 0).astype(jnp.float32)
    jax.lax.fori_loop(0, npoint, body, (dist0, oh0))


def _fps(xT, npoint):
    xyz = jnp.transpose(xT, (0, 2, 1))
    n = xyz.shape[1]

    def single(x):
        def body(i, state):
            centroids, distance, farthest = state
            centroids = centroids.at[i].set(farthest)
            centroid = x[farthest]
            d = jnp.sum((x - centroid) ** 2, -1)
            distance = jnp.minimum(distance, d)
            farthest = jnp.argmax(distance).astype(jnp.int32)
            return (centroids, distance, farthest)
        init = (jnp.zeros((npoint,), dtype=jnp.int32),
                jnp.full((n,), 1e10, dtype=jnp.float32), jnp.int32(0))
        return jax.lax.fori_loop(0, npoint, body, init)[0]

    idx = jax.vmap(single)(xyz)
    return jax.vmap(lambda p, i: p[i])(xyz, idx)


def _group_body(idx_ref, nx_ref, xf_ref, out_ref, *, k, nfeat):
    idx = idx_ref[0]
    nx = nx_ref[0]
    xf = xf_ref[0]
    n = xf.shape[0]
    lane = jax.lax.broadcasted_iota(jnp.int32, (idx.shape[0], n), 1)
    for kk in range(k):
        sel = (lane == idx[:, kk:kk + 1]).astype(jnp.float32)
        g = jnp.dot(sel, xf, precision=jax.lax.Precision.HIGHEST,
                    preferred_element_type=jnp.float32)
        gx = g[:, :3] - nx
        row = jnp.concatenate([gx, g[:, 3:]], axis=-1) if nfeat else gx
        out_ref[0, :, kk, :] = row


def _sqdist_xla(src, dst):
    d = -2.0 * jnp.matmul(src, jnp.swapaxes(dst, 1, 2))
    d = d + jnp.sum(src ** 2, -1)[:, :, None]
    d = d + jnp.sum(dst ** 2, -1)[:, None, :]
    return d


def _group(new_xyz, xyz, xf, radius, k, sblk):
    b, s, _ = new_xyz.shape
    n = xyz.shape[1]
    c = xf.shape[2]
    nfeat = c > 3
    sqrdists = _sqdist_xla(new_xyz, xyz)
    idx = jnp.broadcast_to(jnp.arange(n, dtype=jnp.int32), (b, s, n))
    idx = jnp.where(sqrdists > radius * radius, n, idx)
    idx = jnp.sort(idx, axis=-1)[:, :, :k]
    first = idx[:, :, 0:1]
    idx = jnp.where(idx == n, jnp.broadcast_to(first, idx.shape), idx)
    g = jax.vmap(lambda p, i: p[i])(xf, idx)
    gx = g[..., :3] - new_xyz[:, :, None, :]
    return jnp.concatenate([gx, g[..., 3:]], axis=-1) if nfeat else gx


def _layer_body(x_ref, a_ref, c_ref, wt_ref, b_ref, y_ref, s1_ref, s2_ref,
                *, norm_in):
    x = x_ref[...]
    if norm_in:
        x = jnp.maximum(x * a_ref[...] + c_ref[...], 0.0)
    y = jnp.dot(x, wt_ref[...], preferred_element_type=jnp.float32)
    y = y + b_ref[...]
    y_ref[...] = y
    ps = jnp.sum(y, axis=0, keepdims=True)
    pq = jnp.sum(y * y, axis=0, keepdims=True)

    @pl.when(pl.program_id(0) == 0)
    def _():
        s1_ref[...] = ps
        s2_ref[...] = pq

    @pl.when(pl.program_id(0) != 0)
    def _():
        s1_ref[...] += ps
        s2_ref[...] += pq


def _layer(x, ac, wt, bias):
    p, cin = x.shape
    cout = wt.shape[1]
    blk = min(p, 8192 if cin <= 128 else 2048)
    grid = (p // blk,)
    norm_in = ac is not None
    if ac is None:
        ac = (jnp.ones((1, cin), jnp.float32), jnp.zeros((1, cin), jnp.float32))
    fullspec = pl.BlockSpec((1, cin), lambda i: (0, 0))
    return pl.pallas_call(
        functools.partial(_layer_body, norm_in=norm_in),
        grid=grid,
        in_specs=[
            pl.BlockSpec((blk, cin), lambda i: (i, 0)),
            fullspec, fullspec,
            pl.BlockSpec((cin, cout), lambda i: (0, 0)),
            pl.BlockSpec((1, cout), lambda i: (0, 0)),
        ],
        out_specs=[
            pl.BlockSpec((blk, cout), lambda i: (i, 0)),
            pl.BlockSpec((1, cout), lambda i: (0, 0)),
            pl.BlockSpec((1, cout), lambda i: (0, 0)),
        ],
        out_shape=[
            jax.ShapeDtypeStruct((p, cout), jnp.float32),
            jax.ShapeDtypeStruct((1, cout), jnp.float32),
            jax.ShapeDtypeStruct((1, cout), jnp.float32),
        ],
    )(x, ac[0], ac[1], wt, bias)


def _ac_from_stats(s1, s2, p, g, bt):
    m = s1[0] / p
    v = s2[0] / p - m * m
    inv = g / jnp.sqrt(v + 1e-5)
    return inv.reshape(1, -1), (bt - m * inv).reshape(1, -1)


def _mlp_chain(x, ps):
    ac = None
    y = x
    for (w, b, g, bt) in ps:
        y, s1, s2 = _layer(y, ac, w.T, b.reshape(1, -1))
        ac = _ac_from_stats(s1, s2, float(y.shape[0]), g, bt)
    return y, ac


def _norm_max_body(y_ref, a_ref, c_ref, o_ref):
    z = jnp.maximum(y_ref[...] * a_ref[...] + c_ref[...], 0.0)
    o_ref[...] = jnp.max(z, axis=1)


def _norm_relu_max(y3, ac):
    p, k, c = y3.shape
    blk = min(p, 512)
    return pl.pallas_call(
        _norm_max_body,
        grid=(p // blk,),
        in_specs=[
            pl.BlockSpec((blk, k, c), lambda i: (i, 0, 0)),
            pl.BlockSpec((1, 1, c), lambda i: (0, 0, 0)),
            pl.BlockSpec((1, 1, c), lambda i: (0, 0, 0)),
        ],
        out_specs=pl.BlockSpec((blk, c), lambda i: (i, 0)),
        out_shape=jax.ShapeDtypeStruct((p, c), jnp.float32),
    )(y3, ac[0].reshape(1, 1, c), ac[1].reshape(1, 1, c))


def _norm_body(y_ref, a_ref, c_ref, o_ref):
    o_ref[...] = jnp.maximum(y_ref[...] * a_ref[...] + c_ref[...], 0.0)


def _norm_relu(y, ac):
    p, c = y.shape
    blk = min(p, 4096)
    return pl.pallas_call(
        _norm_body,
        grid=(p // blk,),
        in_specs=[
            pl.BlockSpec((blk, c), lambda i: (i, 0)),
            pl.BlockSpec((1, c), lambda i: (0, 0)),
            pl.BlockSpec((1, c), lambda i: (0, 0)),
        ],
        out_specs=pl.BlockSpec((blk, c), lambda i: (i, 0)),
        out_shape=jax.ShapeDtypeStruct((p, c), jnp.float32),
    )(y, ac[0], ac[1])


def _interp_body(x1_ref, x2T_ref, p2_ref, o_ref):
    x1 = x1_ref[0]
    x2T = x2T_ref[0]
    d = -2.0 * jnp.dot(x1, x2T, preferred_element_type=jnp.float32)
    ss = (x1[:, 0:1] ** 2 + x1[:, 1:2] ** 2) + x1[:, 2:3] ** 2
    d = d + ss
    d = d + jnp.sum(x2T ** 2, axis=0, keepdims=True)
    wacc = jnp.zeros_like(d)
    wsum = jnp.zeros((d.shape[0], 1), jnp.float32)
    for _ in range(3):
        m = jnp.min(d, axis=1, keepdims=True)
        eq = d == m
        fm = jnp.logical_and(eq, _cumsum_lanes(eq.astype(jnp.float32)) == 1.0)
        w = 1.0 / jnp.maximum(m, 1e-10)
        wacc = wacc + w * fm.astype(jnp.float32)
        wsum = wsum + w
        d = jnp.where(fm, 1e30, d)
    wacc = wacc / wsum
    o_ref[0] = jnp.dot(wacc, p2_ref[0], precision=jax.lax.Precision.HIGHEST,
                       preferred_element_type=jnp.float32)


def _wgather_body(idx_ref, w_ref, p2_ref, o_ref):
    idx = idx_ref[0]
    w = w_ref[0]
    p2 = p2_ref[0]
    n2 = p2.shape[0]
    lane = jax.lax.broadcasted_iota(jnp.int32, (idx.shape[0], n2), 1)
    wacc = jnp.zeros((idx.shape[0], n2), jnp.float32)
    for j in range(3):
        wacc = wacc + w[:, j:j + 1] * (lane == idx[:, j:j + 1]).astype(jnp.float32)
    o_ref[0] = jnp.dot(wacc, p2, precision=jax.lax.Precision.HIGHEST,
                       preferred_element_type=jnp.float32)


def _interp(xyz1, xyz2, p2, blk):
    dists = _sqdist_xla(xyz1, xyz2)
    neg, idx = jax.lax.top_k(-dists, 3)
    d3 = jnp.maximum(-neg, 1e-10)
    weight = 1.0 / d3
    weight = weight / jnp.sum(weight, axis=-1, keepdims=True)
    b, n1, _ = xyz1.shape
    n2 = p2.shape[1]
    c2 = p2.shape[2]
    return pl.pallas_call(
        _wgather_body,
        grid=(b, n1 // blk),
        in_specs=[
            pl.BlockSpec((1, blk, 3), lambda i, j: (i, j, 0)),
            pl.BlockSpec((1, blk, 3), lambda i, j: (i, j, 0)),
            pl.BlockSpec((1, n2, c2), lambda i, j: (i, 0, 0)),
        ],
        out_specs=pl.BlockSpec((1, blk, c2), lambda i, j: (i, j, 0)),
        out_shape=jax.ShapeDtypeStruct((b, n1, c2), jnp.float32),
    )(idx, weight, p2)


def _interp_pallas(xyz1, xyz2, p2, blk):
    b, n1, _ = xyz1.shape
    n2 = xyz2.shape[1]
    c2 = p2.shape[2]
    x2T = jnp.transpose(xyz2, (0, 2, 1))
    return pl.pallas_call(
        _interp_body,
        grid=(b, n1 // blk),
        in_specs=[
            pl.BlockSpec((1, blk, 3), lambda i, j: (i, j, 0)),
            pl.BlockSpec((1, 3, n2), lambda i, j: (i, 0, 0)),
            pl.BlockSpec((1, n2, c2), lambda i, j: (i, 0, 0)),
        ],
        out_specs=pl.BlockSpec((1, blk, c2), lambda i, j: (i, j, 0)),
        out_shape=jax.ShapeDtypeStruct((b, n1, c2), jnp.float32),
    )(xyz1, x2T, p2)


def _head_body(y_ref, a_ref, c_ref, wt_ref, b_ref, o_ref):
    z = jnp.maximum(y_ref[...] * a_ref[...] + c_ref[...], 0.0)
    lg = jnp.dot(z, wt_ref[...], preferred_element_type=jnp.float32)
    lg = lg + b_ref[...]
    mx = jnp.max(lg, axis=-1, keepdims=True)
    sh = lg - mx
    lse = jnp.log(jnp.sum(jnp.exp(sh), axis=-1, keepdims=True))
    o_ref[...] = sh - lse


def _head_out(y, ac, w2, b2):
    p, c = y.shape
    nc = w2.shape[0]
    blk = min(p, 2048)
    return pl.pallas_call(
        _head_body,
        grid=(p // blk,),
        in_specs=[
            pl.BlockSpec((blk, c), lambda i: (i, 0)),
            pl.BlockSpec((1, c), lambda i: (0, 0)),
            pl.BlockSpec((1, c), lambda i: (0, 0)),
            pl.BlockSpec((c, nc), lambda i: (0, 0)),
            pl.BlockSpec((1, nc), lambda i: (0, 0)),
        ],
        out_specs=pl.BlockSpec((blk, nc), lambda i: (i, 0)),
        out_shape=jax.ShapeDtypeStruct((p, nc), jnp.float32),
    )(y, ac[0], ac[1], w2.T, b2.reshape(1, -1))


def _set_abstraction(xyz, points, npoint, radius, k, ps, sblk):
    b, n, _ = xyz.shape
    xT = jnp.transpose(xyz, (0, 2, 1))
    new_xyz = _fps(xT, npoint)
    xf = xyz if points is None else jnp.concatenate([xyz, points], axis=-1)
    grouped = _group(new_xyz, xyz, xf, radius, k, sblk)
    c = grouped.shape[-1]
    y, ac = _mlp_chain(grouped.reshape(b * npoint * k, c), ps)
    cout = y.shape[-1]
    out = _norm_relu_max(y.reshape(b * npoint, k, cout), ac)
    return new_xyz, out.reshape(b, npoint, cout)


def _feature_prop(xyz1, xyz2, p1, p2, ps, blk):
    b, n1, _ = xyz1.shape
    interp = _interp(xyz1, xyz2, p2, blk)
    x = interp if p1 is None else jnp.concatenate([p1, interp], axis=-1)
    c = x.shape[-1]
    y, ac = _mlp_chain(x.reshape(b * n1, c), ps)
    cout = y.shape[-1]
    return _norm_relu(y, ac).reshape(b, n1, cout)


def kernel(xyz, params):
    b = xyz.shape[0]
    n = xyz.shape[2]
    l0_xyz = jnp.transpose(xyz, (0, 2, 1))
    l1_xyz, l1_p = _set_abstraction(l0_xyz, None, 1024, 0.1, 32,
                                    params['sa1'], sblk=256)
    l2_xyz, l2_p = _set_abstraction(l1_xyz, l1_p, 256, 0.2, 32,
                                    params['sa2'], sblk=256)
    l3_xyz, l3_p = _set_abstraction(l2_xyz, l2_p, 64, 0.4, 32,
                                    params['sa3'], sblk=64)
    l4_xyz, l4_p = _set_abstraction(l3_xyz, l3_p, 16, 0.8, 32,
                                    params['sa4'], sblk=16)
    l3_p = _feature_prop(l3_xyz, l4_xyz, l3_p, l4_p, params['fp4'], blk=64)
    l2_p = _feature_prop(l2_xyz, l3_xyz, l2_p, l3_p, params['fp3'], blk=256)
    l1_p = _feature_prop(l1_xyz, l2_xyz, l1_p, l2_p, params['fp2'], blk=512)
    l0_p = _feature_prop(l0_xyz, l1_xyz, None, l1_p, params['fp1'], blk=512)
    y, ac = _mlp_chain(l0_p.reshape(b * n, -1), params['head'])
    w2, b2 = params['conv2']
    out = _head_out(y, ac, w2, b2)
    return out.reshape(b, n, -1)
